# baseline (speedup 1.0000x reference)
import functools

import jax
import jax.numpy as jnp
import numpy as np
from jax import lax
from jax.experimental import pallas as pl
from jax.experimental.pallas import tpu as pltpu
from jax.experimental.pallas import tpu_sc as plsc

_NUM_CORES = 2
_NUM_SUBCORES = 16
_NUM_WORKERS = _NUM_CORES * _NUM_SUBCORES
_CHUNK = 80


def _tables_body(x_ref, wa_ref, wb_ref, b1_ref, a_ref, b_ref):
    xv = x_ref[...]
    a_ref[...] = (
        jnp.dot(xv, wa_ref[...], preferred_element_type=jnp.float32) + b1_ref[...]
    ).astype(jnp.bfloat16)
    b_ref[...] = jnp.dot(
        xv, wb_ref[...], preferred_element_type=jnp.float32
    ).astype(jnp.bfloat16)


def _make_tables(x, w1a, w1b, b1):
    n, _ = x.shape
    h = w1a.shape[1]
    return pl.pallas_call(
        _tables_body,
        out_shape=[
            jax.ShapeDtypeStruct((n, h), jnp.bfloat16),
            jax.ShapeDtypeStruct((n, h), jnp.bfloat16),
        ],
    )(x, w1a, w1b, b1.reshape(1, h))


def _gather_add(start, end, tab_a, tab_b):
    e = start.shape[0]
    h = 2 * tab_a.shape[1]
    per_worker = e // _NUM_WORKERS
    chunks = per_worker // _CHUNK
    pairs = chunks // 2
    mesh = plsc.VectorSubcoreMesh(core_axis_name="c", subcore_axis_name="s")

    start2 = start.reshape(e // _CHUNK, _CHUNK)
    end2 = end.reshape(e // _CHUNK, _CHUNK)

    rows_per_chunk = _CHUNK // 2

    @functools.partial(
        pl.kernel,
        mesh=mesh,
        out_type=jax.ShapeDtypeStruct((e // 2, 2 * h), jnp.float32),
        compiler_params=pltpu.CompilerParams(use_tc_tiling_on_sc=False),
        scratch_types=[
            pltpu.VMEM((chunks, _CHUNK), jnp.int32),
            pltpu.VMEM((chunks, _CHUNK), jnp.int32),
            pltpu.VMEM((_CHUNK, h // 2), jnp.int32),
            pltpu.VMEM((_CHUNK, h // 2), jnp.int32),
            pltpu.VMEM((_CHUNK, h // 2), jnp.int32),
            pltpu.VMEM((_CHUNK, h // 2), jnp.int32),
            pltpu.VMEM((rows_per_chunk, 2 * h), jnp.float32),
            pltpu.VMEM((rows_per_chunk, 2 * h), jnp.float32),
            pltpu.SemaphoreType.DMA,
            pltpu.SemaphoreType.DMA,
            pltpu.SemaphoreType.DMA,
            pltpu.SemaphoreType.DMA,
        ],
    )
    def sc_kernel(start_hbm, end_hbm, a_hbm, b_hbm, g_hbm,
                  idx_s, idx_e, a0, b0, a1, b1, o0, o1, sg0, sg1, st0, st1):
        wid = lax.axis_index("s") * _NUM_CORES + lax.axis_index("c")
        cbase = wid * chunks

        def fire(c, ba, bb, sem):
            pltpu.async_copy(a_hbm.at[idx_s.at[c]], ba, sem)
            pltpu.async_copy(b_hbm.at[idx_e.at[c]], bb, sem)

        def wait_g(ba, bb, sem):
            pltpu.make_async_copy(a_hbm.at[idx_s.at[0]], ba, sem).wait()
            pltpu.make_async_copy(b_hbm.at[idx_e.at[0]], bb, sem).wait()

        hi_mask = jnp.int32(-65536)

        def bf16_pair_to_f32(w):
            lo = lax.bitcast_convert_type(lax.shift_left(w, 16), jnp.float32)
            hi = lax.bitcast_convert_type(lax.bitwise_and(w, hi_mask), jnp.float32)
            return lo, hi

        def add_rows(ba, bb, oo):
            def pair_rows(p, c2):
                for half in range(2):
                    for t in range(h // 32):
                        src = pl.ds(t * 16, 16)
                        alo, ahi = bf16_pair_to_f32(ba[2 * p + half, src])
                        blo, bhi = bf16_pair_to_f32(bb[2 * p + half, src])
                        base = half * h + t * 32
                        oo[p, pl.ds(base, 16)] = alo + blo
                        oo[p, pl.ds(base + 16, 16)] = ahi + bhi
                return c2

            lax.fori_loop(0, rows_per_chunk, pair_rows, 0)

        def out_slice(c):
            base = pl.multiple_of((cbase + c) * rows_per_chunk, 8)
            return g_hbm.at[pl.ds(base, rows_per_chunk)]

        def wait_st(oo, sem):
            pltpu.make_async_copy(oo, out_slice(0), sem).wait()

        pltpu.sync_copy(start_hbm.at[pl.ds(cbase, chunks)], idx_s)
        pltpu.sync_copy(end_hbm.at[pl.ds(cbase, chunks)], idx_e)
        fire(0, a0, b0, sg0)

        def pair_body(j, carry):
            c = 2 * j

            @pl.when(j > 0)
            def _():
                wait_st(o0, st0)
                wait_st(o1, st1)

            fire(c + 1, a1, b1, sg1)
            wait_g(a0, b0, sg0)
            add_rows(a0, b0, o0)
            pltpu.async_copy(o0, out_slice(c), st0)
            fire(c + 2, a0, b0, sg0)
            wait_g(a1, b1, sg1)
            add_rows(a1, b1, o1)
            pltpu.async_copy(o1, out_slice(c + 1), st1)
            return carry

        lax.fori_loop(0, pairs, pair_body, 0)

        wait_g(a0, b0, sg0)
        wait_st(o0, st0)
        add_rows(a0, b0, o0)
        pltpu.sync_copy(o0, out_slice(chunks - 1))
        wait_st(o1, st1)

    return sc_kernel(start2, end2, tab_a, tab_b)


def _ln_tanh_cols(v, gain, bias, h):
    hh, block = v.shape
    s = v.reshape(2, h, block)
    mu = jnp.mean(s, axis=1, keepdims=True)
    d = s - jnp.broadcast_to(mu, s.shape)
    var = jnp.mean(d * d, axis=1, keepdims=True)
    r = jnp.broadcast_to(jax.lax.rsqrt(var + 1e-5), s.shape)
    return jnp.tanh((d * r).reshape(hh, block) * gain + bias)


def _head_body(g_ref, g1_ref, be1_ref, w2_ref, b2_ref, g2_ref, be2_ref, w3_ref, b3_ref, out_ref):
    h = g_ref.shape[1] // 2
    v = jnp.transpose(g_ref[...])
    v = _ln_tanh_cols(v, g1_ref[...], be1_ref[...], h)
    v = (
        lax.dot_general(
            w2_ref[...], v, (((0,), (0,)), ((), ())),
            preferred_element_type=jnp.float32,
        )
        + b2_ref[...]
    )
    v = _ln_tanh_cols(v, g2_ref[...], be2_ref[...], h)
    res = (
        lax.dot_general(
            w3_ref[...], v, (((0,), (0,)), ((), ())),
            preferred_element_type=jnp.float32,
        )
        + b3_ref[...]
    )
    out_ref[...] = res[None]


def _edge_head(g2, g1, be1, w2, b2, g2n, be2, w3, b3, block):
    rows, hh = g2.shape
    h = hh // 2
    grid = rows // block
    full = lambda i: (0, 0)

    def dup(v):
        return jnp.concatenate([v, v]).reshape(hh, 1)

    w2d = jnp.zeros((hh, hh), jnp.float32)
    w2d = w2d.at[:h, :h].set(w2).at[h:, h:].set(w2)
    w3d = jnp.zeros((hh, 2), jnp.float32)
    w3d = w3d.at[:h, 0].set(w3[:, 0]).at[h:, 1].set(w3[:, 0])

    out = pl.pallas_call(
        _head_body,
        grid=(grid,),
        in_specs=[
            pl.BlockSpec((block, hh), lambda i: (i, 0)),
            pl.BlockSpec((hh, 1), full),
            pl.BlockSpec((hh, 1), full),
            pl.BlockSpec((hh, hh), full),
            pl.BlockSpec((hh, 1), full),
            pl.BlockSpec((hh, 1), full),
            pl.BlockSpec((hh, 1), full),
            pl.BlockSpec((hh, 2), full),
            pl.BlockSpec((1, 1), full),
        ],
        out_specs=pl.BlockSpec((1, 2, block), lambda i: (i, 0, 0)),
        out_shape=jax.ShapeDtypeStruct((grid, 2, block), jnp.float32),
        compiler_params=pltpu.CompilerParams(
            dimension_semantics=("parallel",),
        ),
    )(
        g2,
        dup(g1),
        dup(be1),
        w2d,
        dup(b2),
        dup(g2n),
        dup(be2),
        w3d,
        b3.reshape(1, 1),
    )
    return jnp.transpose(out, (0, 2, 1))


def _unpack_perm(h):
    perm = np.empty(h, np.int32)
    for gbase in range(0, h, 32):
        for i in range(16):
            perm[gbase + 2 * i] = gbase + i
            perm[gbase + 2 * i + 1] = gbase + 16 + i
    return perm


def kernel(x, edge_index, W1, b1, g1, be1, W2, b2, g2, be2, W3, b3):
    n, d = x.shape
    e = edge_index.shape[1]
    h = W2.shape[0]
    perm = _unpack_perm(h)
    tab_a, tab_b = _make_tables(x, W1[:d][:, perm], W1[d:][:, perm], b1[perm])
    tab_ai = lax.bitcast_convert_type(tab_a.reshape(n, h // 2, 2), jnp.int32)
    tab_bi = lax.bitcast_convert_type(tab_b.reshape(n, h // 2, 2), jnp.int32)
    gpaired = _gather_add(edge_index[0], edge_index[1], tab_ai, tab_bi)
    out = _edge_head(gpaired, g1, be1, W2, b2, g2, be2, W3, b3, block=1280)
    return out.reshape(e)


# speedup vs baseline: 1.1503x; 1.1503x over previous
import functools

import jax
import jax.numpy as jnp
import numpy as np
from jax import lax
from jax.experimental import pallas as pl
from jax.experimental.pallas import tpu as pltpu
from jax.experimental.pallas import tpu_sc as plsc

_NUM_CORES = 2
_NUM_SUBCORES = 16
_NUM_WORKERS = _NUM_CORES * _NUM_SUBCORES
_CHUNK = 80


def _tables_body(x_ref, wa_ref, wb_ref, b1_ref, a_ref, b_ref):
    xv = x_ref[...]
    a_ref[...] = (
        jnp.dot(xv, wa_ref[...], preferred_element_type=jnp.float32) + b1_ref[...]
    )
    b_ref[...] = jnp.dot(xv, wb_ref[...], preferred_element_type=jnp.float32)


def _make_tables(x, w1a, w1b, b1):
    n, _ = x.shape
    h = w1a.shape[1]
    return pl.pallas_call(
        _tables_body,
        out_shape=[
            jax.ShapeDtypeStruct((n, h), jnp.float32),
            jax.ShapeDtypeStruct((n, h), jnp.float32),
        ],
    )(x, w1a, w1b, b1.reshape(1, h))


def _gather_add(start, end, tab_a, tab_b):
    e = start.shape[0]
    h = tab_a.shape[1]
    per_worker = e // _NUM_WORKERS
    chunks = per_worker // _CHUNK
    pairs = chunks // 2
    mesh = plsc.VectorSubcoreMesh(core_axis_name="c", subcore_axis_name="s")

    start2 = start.reshape(e // _CHUNK, _CHUNK)
    end2 = end.reshape(e // _CHUNK, _CHUNK)

    rows_per_chunk = _CHUNK // 2

    @functools.partial(
        pl.kernel,
        mesh=mesh,
        out_type=jax.ShapeDtypeStruct((e // 2, 2 * h), jnp.float32),
        compiler_params=pltpu.CompilerParams(use_tc_tiling_on_sc=False),
        scratch_types=[
            pltpu.VMEM((chunks, _CHUNK), jnp.int32),
            pltpu.VMEM((chunks, _CHUNK), jnp.int32),
            pltpu.VMEM((_CHUNK, h), jnp.float32),
            pltpu.VMEM((_CHUNK, h), jnp.float32),
            pltpu.VMEM((_CHUNK, h), jnp.float32),
            pltpu.VMEM((_CHUNK, h), jnp.float32),
            pltpu.VMEM((rows_per_chunk, 2 * h), jnp.float32),
            pltpu.VMEM((rows_per_chunk, 2 * h), jnp.float32),
            pltpu.SemaphoreType.DMA,
            pltpu.SemaphoreType.DMA,
            pltpu.SemaphoreType.DMA,
            pltpu.SemaphoreType.DMA,
        ],
    )
    def sc_kernel(start_hbm, end_hbm, a_hbm, b_hbm, g_hbm,
                  idx_s, idx_e, a0, b0, a1, b1, o0, o1, sg0, sg1, st0, st1):
        wid = lax.axis_index("s") * _NUM_CORES + lax.axis_index("c")
        cbase = wid * chunks

        def fire(c, ba, bb, sem):
            pltpu.async_copy(a_hbm.at[idx_s.at[c]], ba, sem)
            pltpu.async_copy(b_hbm.at[idx_e.at[c]], bb, sem)

        def wait_g(ba, bb, sem):
            pltpu.make_async_copy(a_hbm.at[idx_s.at[0]], ba, sem).wait()
            pltpu.make_async_copy(b_hbm.at[idx_e.at[0]], bb, sem).wait()

        def add_rows(ba, bb, oo):
            def pair_rows(p, c2):
                for half in range(2):
                    for t in range(h // 16):
                        src = pl.ds(t * 16, 16)
                        dst = pl.ds(half * h + t * 16, 16)
                        oo[p, dst] = ba[2 * p + half, src] + bb[2 * p + half, src]
                return c2

            lax.fori_loop(0, rows_per_chunk, pair_rows, 0)

        def out_slice(c):
            base = pl.multiple_of((cbase + c) * rows_per_chunk, 8)
            return g_hbm.at[pl.ds(base, rows_per_chunk)]

        def wait_st(oo, sem):
            pltpu.make_async_copy(oo, out_slice(0), sem).wait()

        pltpu.sync_copy(start_hbm.at[pl.ds(cbase, chunks)], idx_s)
        pltpu.sync_copy(end_hbm.at[pl.ds(cbase, chunks)], idx_e)
        fire(0, a0, b0, sg0)

        def pair_body(j, carry):
            c = 2 * j

            @pl.when(j > 0)
            def _():
                wait_st(o0, st0)
                wait_st(o1, st1)

            fire(c + 1, a1, b1, sg1)
            wait_g(a0, b0, sg0)
            add_rows(a0, b0, o0)
            pltpu.async_copy(o0, out_slice(c), st0)
            fire(c + 2, a0, b0, sg0)
            wait_g(a1, b1, sg1)
            add_rows(a1, b1, o1)
            pltpu.async_copy(o1, out_slice(c + 1), st1)
            return carry

        lax.fori_loop(0, pairs, pair_body, 0)

        wait_g(a0, b0, sg0)
        wait_st(o0, st0)
        add_rows(a0, b0, o0)
        pltpu.sync_copy(o0, out_slice(chunks - 1))
        wait_st(o1, st1)

    return sc_kernel(start2, end2, tab_a, tab_b)


def _ln_tanh_cols(v, gain, bias, h):
    hh, block = v.shape
    s = v.reshape(2, h, block)
    mu = jnp.mean(s, axis=1, keepdims=True)
    d = s - jnp.broadcast_to(mu, s.shape)
    var = jnp.mean(d * d, axis=1, keepdims=True)
    r = jnp.broadcast_to(jax.lax.rsqrt(var + 1e-5), s.shape)
    return jnp.tanh((d * r).reshape(hh, block) * gain + bias)


def _head_body(g_ref, g1_ref, be1_ref, w2_ref, b2_ref, g2_ref, be2_ref, w3_ref, b3_ref, out_ref):
    h = g_ref.shape[1] // 2
    v = jnp.transpose(g_ref[...])
    v = _ln_tanh_cols(v, g1_ref[...], be1_ref[...], h)
    v = (
        lax.dot_general(
            w2_ref[...], v, (((0,), (0,)), ((), ())),
            preferred_element_type=jnp.float32,
        )
        + b2_ref[...]
    )
    v = _ln_tanh_cols(v, g2_ref[...], be2_ref[...], h)
    res = (
        lax.dot_general(
            w3_ref[...], v, (((0,), (0,)), ((), ())),
            preferred_element_type=jnp.float32,
        )
        + b3_ref[...]
    )
    out_ref[...] = res[None]


def _edge_head(g2, g1, be1, w2, b2, g2n, be2, w3, b3, block):
    rows, hh = g2.shape
    h = hh // 2
    grid = rows // block
    full = lambda i: (0, 0)

    def dup(v):
        return jnp.concatenate([v, v]).reshape(hh, 1)

    w2d = jnp.zeros((hh, hh), jnp.float32)
    w2d = w2d.at[:h, :h].set(w2).at[h:, h:].set(w2)
    w3d = jnp.zeros((hh, 2), jnp.float32)
    w3d = w3d.at[:h, 0].set(w3[:, 0]).at[h:, 1].set(w3[:, 0])

    out = pl.pallas_call(
        _head_body,
        grid=(grid,),
        in_specs=[
            pl.BlockSpec((block, hh), lambda i: (i, 0)),
            pl.BlockSpec((hh, 1), full),
            pl.BlockSpec((hh, 1), full),
            pl.BlockSpec((hh, hh), full),
            pl.BlockSpec((hh, 1), full),
            pl.BlockSpec((hh, 1), full),
            pl.BlockSpec((hh, 1), full),
            pl.BlockSpec((hh, 2), full),
            pl.BlockSpec((1, 1), full),
        ],
        out_specs=pl.BlockSpec((1, 2, block), lambda i: (i, 0, 0)),
        out_shape=jax.ShapeDtypeStruct((grid, 2, block), jnp.float32),
        compiler_params=pltpu.CompilerParams(
            dimension_semantics=("parallel",),
        ),
    )(
        g2,
        dup(g1),
        dup(be1),
        w2d,
        dup(b2),
        dup(g2n),
        dup(be2),
        w3d,
        b3.reshape(1, 1),
    )
    return jnp.transpose(out, (0, 2, 1))


def kernel(x, edge_index, W1, b1, g1, be1, W2, b2, g2, be2, W3, b3):
    n, d = x.shape
    e = edge_index.shape[1]
    h = W2.shape[0]
    tab_a, tab_b = _make_tables(x, W1[:d], W1[d:], b1)
    gpaired = _gather_add(edge_index[0], edge_index[1], tab_a, tab_b)
    out = _edge_head(gpaired, g1, be1, W2, b2, g2, be2, W3, b3, block=1280)
    return out.reshape(e)
